# baseline (speedup 1.0000x reference)
import functools

import jax
import jax.numpy as jnp
from jax import lax
from jax.experimental import pallas as pl
from jax.experimental.pallas import tpu as pltpu
from jax.experimental.pallas import tpu_sc as plsc

D = 64
B = 4096
_STRIPE = 128
_NSB = 4

_info = plsc.get_sparse_core_info()
_NC = _info.num_cores
_NS = _info.num_subcores
_NW = _NC * _NS
_BPW = B // _NW

_mesh = plsc.VectorSubcoreMesh(core_axis_name="c", subcore_axis_name="s")


def _stage_idx(idx_hbm, iv, base):
    pltpu.sync_copy(idx_hbm.at[pl.ds(base, _BPW)], iv)


def _fire_rows(tab, iv, rv, sem):
    def body(g, carry):
        v = iv[pl.ds(g * 16, 16)]
        for j in range(16):
            row = v[j]
            pltpu.async_copy(tab.at[row], rv.at[g * 16 + j], sem)
        return carry

    lax.fori_loop(0, _BPW // 16, body, 0)


def _drain_rows(out_slice, rv, sem):
    pltpu.make_async_copy(out_slice, rv, sem).wait()
    pltpu.sync_copy(rv, out_slice)


@functools.partial(
    pl.kernel,
    mesh=_mesh,
    compiler_params=pltpu.CompilerParams(needs_layout_passes=False),
    out_type=jax.ShapeDtypeStruct((B, D), jnp.float32),
    scratch_types=(
        [pltpu.VMEM((_BPW,), jnp.int32),
         pltpu.VMEM((_BPW, D), jnp.float32)]
        + [pltpu.VMEM((D, _STRIPE), jnp.float32) for _ in range(_NSB)]
        + [pltpu.SemaphoreType.DMA for _ in range(_NSB)]
    ),
)
def _gather_a(i_o, t_ot, out_hbm, iv, rv, sb0, sb1, sb2, sb3, q0, q1, q2, q3):
    wid = lax.axis_index("s") * _NC + lax.axis_index("c")
    base = wid * _BPW
    sbufs = (sb0, sb1, sb2, sb3)
    qsems = (q0, q1, q2, q3)
    _stage_idx(i_o, iv, base)

    jvecs = [lax.iota(jnp.int32, 16) + 16 * k for k in range(4)]

    def _extract(lane, buf, i):
        lvec = jnp.full((16,), lane, dtype=jnp.int32)
        for k in range(4):
            col = plsc.load_gather(buf, [jvecs[k], lvec])
            rv[i, pl.ds(k * 16, 16)] = col

    def _stripe_body(g, carry):
        v = iv[pl.ds(g * 16, 16)]
        pend = []
        for j in range(16):
            row = v[j]
            base_lane = pl.multiple_of((row // _STRIPE) * _STRIPE, _STRIPE)
            lane = row - base_lane
            nb = j % _NSB
            if j >= _NSB:
                plane, pcopy = pend[j - _NSB]
                pcopy.wait()
                _extract(plane, sbufs[nb], g * 16 + (j - _NSB))
            cp = pltpu.async_copy(
                t_ot.at[:, pl.ds(base_lane, _STRIPE)], sbufs[nb], qsems[nb])
            pend.append((lane, cp))
        for j in range(16 - _NSB, 16):
            plane, pcopy = pend[j]
            pcopy.wait()
            _extract(plane, sbufs[j % _NSB], g * 16 + j)
        return carry

    lax.fori_loop(0, _BPW // 16, _stripe_body, 0)
    pltpu.sync_copy(rv, out_hbm.at[pl.ds(base, _BPW)])


@functools.partial(
    pl.kernel,
    mesh=_mesh,
    compiler_params=pltpu.CompilerParams(needs_layout_passes=False),
    out_type=jax.ShapeDtypeStruct((4, B, _STRIPE), jnp.float32),
    scratch_types=(
        [pltpu.VMEM((_BPW,), jnp.int32) for _ in range(4)]
        + [pltpu.VMEM((_BPW, _STRIPE), jnp.float32) for _ in range(4)]
        + [pltpu.SemaphoreType.DMA for _ in range(4)]
    ),
)
def _gather_b(i_c, i_cs, i_dp, i_bt, t_c, t_cs, t_dp, t_bt, order_token,
              out_hbm, x0, x1, x2, x3, r0, r1, r2, r3, s0, s1, s2, s3):
    del order_token
    wid = lax.axis_index("s") * _NC + lax.axis_index("c")
    base = wid * _BPW
    idxs = (i_c, i_cs, i_dp, i_bt)
    tabs = (t_c, t_cs, t_dp, t_bt)
    ivs = (x0, x1, x2, x3)
    rvs = (r0, r1, r2, r3)
    sems = (s0, s1, s2, s3)
    for f in range(4):
        _stage_idx(idxs[f], ivs[f], base)
    copies = [
        pltpu.async_copy(tabs[f].at[ivs[f]], rvs[f], sems[f]) for f in range(4)
    ]
    for f in range(4):
        copies[f].wait()
        pltpu.sync_copy(rvs[f], out_hbm.at[f, pl.ds(base, _BPW)])


_BLK = 512


def _proj_body(emb_o_ref, emb_b_ref, w_ref, b_ref, out_ref):
    x_o = emb_o_ref[...]
    eb = emb_b_ref[...]
    x_c, x_cs, x_dp, x_bt = (eb[0, :, :D], eb[1, :, :D],
                             eb[2, :, :D], eb[3, :, :D])
    w = w_ref[...]
    bias = b_ref[...]
    x_cs_t = jnp.swapaxes(x_cs, 0, 1)
    x_dp_t = jnp.swapaxes(x_dp, 0, 1)
    x_bt_t = jnp.swapaxes(x_bt, 0, 1)
    cdims = (((1,), (1,)), ((), ()))
    y_c_t = (lax.dot_general(w[0], x_c, cdims,
                             preferred_element_type=jnp.float32)
             + bias[0][:, None] + x_cs_t)
    y_o_t = (lax.dot_general(w[1], x_o, cdims,
                             preferred_element_type=jnp.float32)
             + bias[1][:, None] + x_dp_t)
    y_dp_t = (lax.dot_general(w[2], x_dp, cdims,
                              preferred_element_type=jnp.float32)
              + bias[2][:, None] + x_bt_t)
    out_ref[...] = jnp.concatenate([y_c_t, x_cs_t, y_o_t, y_dp_t, x_bt_t],
                                   axis=0)


_proj = pl.pallas_call(
    _proj_body,
    grid=(B // _BLK,),
    in_specs=[
        pl.BlockSpec((_BLK, D), lambda i: (i, 0)),
        pl.BlockSpec((4, _BLK, _STRIPE), lambda i: (0, i, 0)),
        pl.BlockSpec((3, D, D), lambda i: (0, 0, 0)),
        pl.BlockSpec((3, D), lambda i: (0, 0)),
    ],
    out_specs=pl.BlockSpec((5 * D, _BLK), lambda i: (0, i)),
    out_shape=jax.ShapeDtypeStruct((5 * D, B), jnp.float32),
)


def kernel(campaignid, campaignsetid, offerid, demand_pkgname, business_type,
           table_campaignid, table_campaignsetid, table_offerid,
           table_demand_pkgname, table_business_type,
           W_campaignid, b_campaignid, W_offerid, b_offerid,
           W_demand_pkgname, b_demand_pkgname):
    i_c = campaignid.astype(jnp.int32)
    i_cs = campaignsetid.astype(jnp.int32)
    i_o = offerid.astype(jnp.int32)
    i_dp = demand_pkgname.astype(jnp.int32)
    i_bt = business_type.astype(jnp.int32)
    emb_o = _gather_a(i_o, table_offerid.T)
    pad = lambda t: jnp.pad(t, ((0, 0), (0, _STRIPE - D)))
    emb_b = _gather_b(i_c, i_cs, i_dp, i_bt,
                      pad(table_campaignid), pad(table_campaignsetid),
                      pad(table_demand_pkgname), pad(table_business_type),
                      emb_o)
    w = jnp.stack([W_campaignid, W_offerid, W_demand_pkgname])
    bias = jnp.stack([b_campaignid, b_offerid, b_demand_pkgname])
    return _proj(emb_o, emb_b, w, bias).T


# speedup vs baseline: 1.4306x; 1.4306x over previous
import functools

import jax
import jax.numpy as jnp
from jax import lax
from jax.experimental import pallas as pl
from jax.experimental.pallas import tpu as pltpu
from jax.experimental.pallas import tpu_sc as plsc

D = 64
B = 4096
_STRIPE = 128
_NSB = 4

_info = plsc.get_sparse_core_info()
_NC = _info.num_cores
_NS = _info.num_subcores
_NW = _NC * _NS
_BPW = B // _NW

_mesh = plsc.VectorSubcoreMesh(core_axis_name="c", subcore_axis_name="s")


def _stage_idx(idx_hbm, iv, base):
    pltpu.sync_copy(idx_hbm.at[pl.ds(base, _BPW)], iv)


def _fire_rows(tab, iv, rv, sem):
    def body(g, carry):
        v = iv[pl.ds(g * 16, 16)]
        for j in range(16):
            row = v[j]
            pltpu.async_copy(tab.at[row], rv.at[g * 16 + j], sem)
        return carry

    lax.fori_loop(0, _BPW // 16, body, 0)


def _drain_rows(out_slice, rv, sem):
    pltpu.make_async_copy(out_slice, rv, sem).wait()
    pltpu.sync_copy(rv, out_slice)


@functools.partial(
    pl.kernel,
    mesh=_mesh,
    compiler_params=pltpu.CompilerParams(needs_layout_passes=False),
    out_type=jax.ShapeDtypeStruct((B, D), jnp.float32),
    scratch_types=(
        [pltpu.VMEM((_BPW,), jnp.int32),
         pltpu.VMEM((_BPW, D), jnp.float32)]
        + [pltpu.VMEM((D, _STRIPE), jnp.float32) for _ in range(_NSB)]
        + [pltpu.SemaphoreType.DMA for _ in range(_NSB)]
    ),
)
def _gather_a(i_o, t_ot, out_hbm, iv, rv, sb0, sb1, sb2, sb3, q0, q1, q2, q3):
    wid = lax.axis_index("s") * _NC + lax.axis_index("c")
    base = wid * _BPW
    sbufs = (sb0, sb1, sb2, sb3)
    qsems = (q0, q1, q2, q3)
    _stage_idx(i_o, iv, base)

    jvecs = [lax.iota(jnp.int32, 16) + 16 * k for k in range(4)]

    def _extract(lane, buf, i):
        lvec = jnp.full((16,), lane, dtype=jnp.int32)
        for k in range(4):
            col = plsc.load_gather(buf, [jvecs[k], lvec])
            rv[i, pl.ds(k * 16, 16)] = col

    def _stripe_body(g, carry):
        v = iv[pl.ds(g * 16, 16)]
        pend = []
        for j in range(16):
            row = v[j]
            base_lane = pl.multiple_of((row // _STRIPE) * _STRIPE, _STRIPE)
            lane = row - base_lane
            nb = j % _NSB
            if j >= _NSB:
                plane, pcopy = pend[j - _NSB]
                pcopy.wait()
                _extract(plane, sbufs[nb], g * 16 + (j - _NSB))
            cp = pltpu.async_copy(
                t_ot.at[:, pl.ds(base_lane, _STRIPE)], sbufs[nb], qsems[nb])
            pend.append((lane, cp))
        for j in range(16 - _NSB, 16):
            plane, pcopy = pend[j]
            pcopy.wait()
            _extract(plane, sbufs[j % _NSB], g * 16 + j)
        return carry

    lax.fori_loop(0, _BPW // 16, _stripe_body, 0)
    pltpu.sync_copy(rv, out_hbm.at[pl.ds(base, _BPW)])


@functools.partial(
    pl.kernel,
    mesh=_mesh,
    compiler_params=pltpu.CompilerParams(needs_layout_passes=False),
    out_type=jax.ShapeDtypeStruct((4, B, D), jnp.float32),
    scratch_types=(
        [pltpu.VMEM((_BPW,), jnp.int32) for _ in range(4)]
        + [pltpu.VMEM((_BPW, D), jnp.float32) for _ in range(4)]
        + [pltpu.SemaphoreType.DMA for _ in range(4)]
    ),
)
def _gather_b(i_c, i_cs, i_dp, i_bt, t_c, t_cs, t_dp, t_bt, order_token,
              out_hbm, x0, x1, x2, x3, r0, r1, r2, r3, s0, s1, s2, s3):
    del order_token
    wid = lax.axis_index("s") * _NC + lax.axis_index("c")
    base = wid * _BPW
    idxs = (i_c, i_cs, i_dp, i_bt)
    tabs = (t_c, t_cs, t_dp, t_bt)
    ivs = (x0, x1, x2, x3)
    rvs = (r0, r1, r2, r3)
    sems = (s0, s1, s2, s3)
    for f in range(4):
        _stage_idx(idxs[f], ivs[f], base)
    for f in range(4):
        _fire_rows(tabs[f], ivs[f], rvs[f], sems[f])
    for f in range(4):
        _drain_rows(out_hbm.at[f, pl.ds(base, _BPW)], rvs[f], sems[f])


_BLK = 512


def _proj_body(emb_o_ref, emb_b_ref, w_ref, b_ref, out_ref):
    x_o = emb_o_ref[...]
    eb = emb_b_ref[...]
    x_c, x_cs, x_dp, x_bt = eb[0], eb[1], eb[2], eb[3]
    w = w_ref[...]
    bias = b_ref[...]
    x_cs_t = jnp.swapaxes(x_cs, 0, 1)
    x_dp_t = jnp.swapaxes(x_dp, 0, 1)
    x_bt_t = jnp.swapaxes(x_bt, 0, 1)
    cdims = (((1,), (1,)), ((), ()))
    y_c_t = (lax.dot_general(w[0], x_c, cdims,
                             preferred_element_type=jnp.float32)
             + bias[0][:, None] + x_cs_t)
    y_o_t = (lax.dot_general(w[1], x_o, cdims,
                             preferred_element_type=jnp.float32)
             + bias[1][:, None] + x_dp_t)
    y_dp_t = (lax.dot_general(w[2], x_dp, cdims,
                              preferred_element_type=jnp.float32)
              + bias[2][:, None] + x_bt_t)
    out_ref[...] = jnp.concatenate([y_c_t, x_cs_t, y_o_t, y_dp_t, x_bt_t],
                                   axis=0)


_proj = pl.pallas_call(
    _proj_body,
    grid=(B // _BLK,),
    in_specs=[
        pl.BlockSpec((_BLK, D), lambda i: (i, 0)),
        pl.BlockSpec((4, _BLK, D), lambda i: (0, i, 0)),
        pl.BlockSpec((3, D, D), lambda i: (0, 0, 0)),
        pl.BlockSpec((3, D), lambda i: (0, 0)),
    ],
    out_specs=pl.BlockSpec((5 * D, _BLK), lambda i: (0, i)),
    out_shape=jax.ShapeDtypeStruct((5 * D, B), jnp.float32),
)


def kernel(campaignid, campaignsetid, offerid, demand_pkgname, business_type,
           table_campaignid, table_campaignsetid, table_offerid,
           table_demand_pkgname, table_business_type,
           W_campaignid, b_campaignid, W_offerid, b_offerid,
           W_demand_pkgname, b_demand_pkgname):
    i_c = campaignid.astype(jnp.int32)
    i_cs = campaignsetid.astype(jnp.int32)
    i_o = offerid.astype(jnp.int32)
    i_dp = demand_pkgname.astype(jnp.int32)
    i_bt = business_type.astype(jnp.int32)
    emb_o = _gather_a(i_o, table_offerid.T)
    emb_b = _gather_b(i_c, i_cs, i_dp, i_bt,
                      table_campaignid, table_campaignsetid,
                      table_demand_pkgname, table_business_type, emb_o)
    w = jnp.stack([W_campaignid, W_offerid, W_demand_pkgname])
    bias = jnp.stack([b_campaignid, b_offerid, b_demand_pkgname])
    return _proj(emb_o, emb_b, w, bias).T


# speedup vs baseline: 1.4317x; 1.0008x over previous
import functools

import jax
import jax.numpy as jnp
from jax import lax
from jax.experimental import pallas as pl
from jax.experimental.pallas import tpu as pltpu
from jax.experimental.pallas import tpu_sc as plsc

D = 64
B = 4096
_STRIPE = 128
_NSB = 8

_info = plsc.get_sparse_core_info()
_NC = _info.num_cores
_NS = _info.num_subcores
_NW = _NC * _NS
_BPW = B // _NW

_mesh = plsc.VectorSubcoreMesh(core_axis_name="c", subcore_axis_name="s")


def _stage_idx(idx_hbm, iv, base):
    pltpu.sync_copy(idx_hbm.at[pl.ds(base, _BPW)], iv)


def _fire_rows(tab, iv, rv, sem):
    def body(g, carry):
        v = iv[pl.ds(g * 16, 16)]
        for j in range(16):
            row = v[j]
            pltpu.async_copy(tab.at[row], rv.at[g * 16 + j], sem)
        return carry

    lax.fori_loop(0, _BPW // 16, body, 0)


def _drain_rows(out_slice, rv, sem):
    pltpu.make_async_copy(out_slice, rv, sem).wait()
    pltpu.sync_copy(rv, out_slice)


@functools.partial(
    pl.kernel,
    mesh=_mesh,
    compiler_params=pltpu.CompilerParams(needs_layout_passes=False),
    out_type=jax.ShapeDtypeStruct((B, D), jnp.float32),
    scratch_types=(
        [pltpu.VMEM((_BPW,), jnp.int32),
         pltpu.VMEM((_BPW, D), jnp.float32)]
        + [pltpu.VMEM((D, _STRIPE), jnp.float32) for _ in range(_NSB)]
        + [pltpu.SemaphoreType.DMA for _ in range(_NSB)]
    ),
)
def _gather_a(i_o, t_ot, out_hbm, iv, rv, sb0, sb1, sb2, sb3, sb4, sb5, sb6,
              sb7, q0, q1, q2, q3, q4, q5, q6, q7):
    wid = lax.axis_index("s") * _NC + lax.axis_index("c")
    base = wid * _BPW
    sbufs = (sb0, sb1, sb2, sb3, sb4, sb5, sb6, sb7)
    qsems = (q0, q1, q2, q3, q4, q5, q6, q7)
    _stage_idx(i_o, iv, base)

    jvecs = [lax.iota(jnp.int32, 16) + 16 * k for k in range(4)]

    def _extract(lane, buf, i):
        lvec = jnp.full((16,), lane, dtype=jnp.int32)
        for k in range(4):
            col = plsc.load_gather(buf, [jvecs[k], lvec])
            rv[i, pl.ds(k * 16, 16)] = col

    def _stripe_body(g, carry):
        v = iv[pl.ds(g * 16, 16)]
        pend = []
        for j in range(16):
            row = v[j]
            base_lane = pl.multiple_of((row // _STRIPE) * _STRIPE, _STRIPE)
            lane = row - base_lane
            nb = j % _NSB
            if j >= _NSB:
                plane, pcopy = pend[j - _NSB]
                pcopy.wait()
                _extract(plane, sbufs[nb], g * 16 + (j - _NSB))
            cp = pltpu.async_copy(
                t_ot.at[:, pl.ds(base_lane, _STRIPE)], sbufs[nb], qsems[nb])
            pend.append((lane, cp))
        for j in range(16 - _NSB, 16):
            plane, pcopy = pend[j]
            pcopy.wait()
            _extract(plane, sbufs[j % _NSB], g * 16 + j)
        return carry

    lax.fori_loop(0, _BPW // 16, _stripe_body, 0)
    pltpu.sync_copy(rv, out_hbm.at[pl.ds(base, _BPW)])


@functools.partial(
    pl.kernel,
    mesh=_mesh,
    compiler_params=pltpu.CompilerParams(needs_layout_passes=False),
    out_type=jax.ShapeDtypeStruct((4, B, D), jnp.float32),
    scratch_types=(
        [pltpu.VMEM((_BPW,), jnp.int32) for _ in range(4)]
        + [pltpu.VMEM((_BPW, D), jnp.float32) for _ in range(4)]
        + [pltpu.SemaphoreType.DMA for _ in range(4)]
    ),
)
def _gather_b(i_c, i_cs, i_dp, i_bt, t_c, t_cs, t_dp, t_bt, order_token,
              out_hbm, x0, x1, x2, x3, r0, r1, r2, r3, s0, s1, s2, s3):
    del order_token
    wid = lax.axis_index("s") * _NC + lax.axis_index("c")
    base = wid * _BPW
    idxs = (i_c, i_cs, i_dp, i_bt)
    tabs = (t_c, t_cs, t_dp, t_bt)
    ivs = (x0, x1, x2, x3)
    rvs = (r0, r1, r2, r3)
    sems = (s0, s1, s2, s3)
    for f in range(4):
        _stage_idx(idxs[f], ivs[f], base)
    for f in range(4):
        _fire_rows(tabs[f], ivs[f], rvs[f], sems[f])
    for f in range(4):
        _drain_rows(out_hbm.at[f, pl.ds(base, _BPW)], rvs[f], sems[f])


_BLK = 512


def _proj_body(emb_o_ref, emb_b_ref, w_ref, b_ref, out_ref):
    x_o = emb_o_ref[...]
    eb = emb_b_ref[...]
    x_c, x_cs, x_dp, x_bt = eb[0], eb[1], eb[2], eb[3]
    w = w_ref[...]
    bias = b_ref[...]
    x_cs_t = jnp.swapaxes(x_cs, 0, 1)
    x_dp_t = jnp.swapaxes(x_dp, 0, 1)
    x_bt_t = jnp.swapaxes(x_bt, 0, 1)
    cdims = (((1,), (1,)), ((), ()))
    y_c_t = (lax.dot_general(w[0], x_c, cdims,
                             preferred_element_type=jnp.float32)
             + bias[0][:, None] + x_cs_t)
    y_o_t = (lax.dot_general(w[1], x_o, cdims,
                             preferred_element_type=jnp.float32)
             + bias[1][:, None] + x_dp_t)
    y_dp_t = (lax.dot_general(w[2], x_dp, cdims,
                              preferred_element_type=jnp.float32)
              + bias[2][:, None] + x_bt_t)
    out_ref[...] = jnp.concatenate([y_c_t, x_cs_t, y_o_t, y_dp_t, x_bt_t],
                                   axis=0)


_proj = pl.pallas_call(
    _proj_body,
    grid=(B // _BLK,),
    in_specs=[
        pl.BlockSpec((_BLK, D), lambda i: (i, 0)),
        pl.BlockSpec((4, _BLK, D), lambda i: (0, i, 0)),
        pl.BlockSpec((3, D, D), lambda i: (0, 0, 0)),
        pl.BlockSpec((3, D), lambda i: (0, 0)),
    ],
    out_specs=pl.BlockSpec((5 * D, _BLK), lambda i: (0, i)),
    out_shape=jax.ShapeDtypeStruct((5 * D, B), jnp.float32),
)


def kernel(campaignid, campaignsetid, offerid, demand_pkgname, business_type,
           table_campaignid, table_campaignsetid, table_offerid,
           table_demand_pkgname, table_business_type,
           W_campaignid, b_campaignid, W_offerid, b_offerid,
           W_demand_pkgname, b_demand_pkgname):
    i_c = campaignid.astype(jnp.int32)
    i_cs = campaignsetid.astype(jnp.int32)
    i_o = offerid.astype(jnp.int32)
    i_dp = demand_pkgname.astype(jnp.int32)
    i_bt = business_type.astype(jnp.int32)
    emb_o = _gather_a(i_o, table_offerid.T)
    emb_b = _gather_b(i_c, i_cs, i_dp, i_bt,
                      table_campaignid, table_campaignsetid,
                      table_demand_pkgname, table_business_type, emb_o)
    w = jnp.stack([W_campaignid, W_offerid, W_demand_pkgname])
    bias = jnp.stack([b_campaignid, b_offerid, b_demand_pkgname])
    return _proj(emb_o, emb_b, w, bias).T


# speedup vs baseline: 1.4517x; 1.0140x over previous
import functools

import jax
import jax.numpy as jnp
from jax import lax
from jax.experimental import pallas as pl
from jax.experimental.pallas import tpu as pltpu
from jax.experimental.pallas import tpu_sc as plsc

D = 64
B = 4096
_STRIPE = 128
_NSB = 8

_info = plsc.get_sparse_core_info()
_NC = _info.num_cores
_NS = _info.num_subcores
_NW = _NC * _NS
_BPW = B // _NW

_mesh = plsc.VectorSubcoreMesh(core_axis_name="c", subcore_axis_name="s")


def _stage_idx(idx_hbm, iv, base):
    pltpu.sync_copy(idx_hbm.at[pl.ds(base, _BPW)], iv)


def _fire_rows(tab, iv, rv, sem):
    def body(g, carry):
        v = iv[pl.ds(g * 16, 16)]
        for j in range(16):
            row = v[j]
            pltpu.async_copy(tab.at[row], rv.at[g * 16 + j], sem)
        return carry

    lax.fori_loop(0, _BPW // 16, body, 0)


def _drain_rows(out_slice, rv, sem):
    pltpu.make_async_copy(out_slice, rv, sem).wait()
    pltpu.sync_copy(rv, out_slice)


@functools.partial(
    pl.kernel,
    mesh=_mesh,
    compiler_params=pltpu.CompilerParams(needs_layout_passes=False),
    out_type=jax.ShapeDtypeStruct((B, D), jnp.float32),
    scratch_types=(
        [pltpu.VMEM((_BPW,), jnp.int32),
         pltpu.VMEM((_BPW, D), jnp.float32)]
        + [pltpu.VMEM((D, _STRIPE), jnp.float32) for _ in range(_NSB)]
        + [pltpu.SemaphoreType.DMA for _ in range(_NSB)]
    ),
)
def _gather_a(i_o, t_ot, out_hbm, iv, rv, sb0, sb1, sb2, sb3, sb4, sb5, sb6,
              sb7, q0, q1, q2, q3, q4, q5, q6, q7):
    wid = lax.axis_index("s") * _NC + lax.axis_index("c")
    base = wid * _BPW
    sbufs = (sb0, sb1, sb2, sb3, sb4, sb5, sb6, sb7)
    qsems = (q0, q1, q2, q3, q4, q5, q6, q7)
    _stage_idx(i_o, iv, base)

    jvecs = [lax.iota(jnp.int32, 16) + 16 * k for k in range(4)]

    def _extract(lane, buf, i):
        lvec = jnp.full((16,), lane, dtype=jnp.int32)
        for k in range(4):
            col = plsc.load_gather(buf, [jvecs[k], lvec])
            rv[i, pl.ds(k * 16, 16)] = col

    def _stripe_body(g, carry):
        v = iv[pl.ds(g * 16, 16)]
        pend = []
        for j in range(16):
            row = v[j]
            base_lane = pl.multiple_of((row // _STRIPE) * _STRIPE, _STRIPE)
            lane = row - base_lane
            nb = j % _NSB
            if j >= _NSB:
                plane, pcopy = pend[j - _NSB]
                pcopy.wait()
                _extract(plane, sbufs[nb], g * 16 + (j - _NSB))
            cp = pltpu.async_copy(
                t_ot.at[:, pl.ds(base_lane, _STRIPE)], sbufs[nb], qsems[nb])
            pend.append((lane, cp))
        for j in range(16 - _NSB, 16):
            plane, pcopy = pend[j]
            pcopy.wait()
            _extract(plane, sbufs[j % _NSB], g * 16 + j)
        return carry

    lax.fori_loop(0, _BPW // 16, _stripe_body, 0)
    pltpu.sync_copy(rv, out_hbm.at[pl.ds(base, _BPW)])


@functools.partial(
    pl.kernel,
    mesh=_mesh,
    compiler_params=pltpu.CompilerParams(needs_layout_passes=False),
    out_type=jax.ShapeDtypeStruct((4, B, D), jnp.float32),
    scratch_types=(
        [pltpu.VMEM((_BPW,), jnp.int32) for _ in range(4)]
        + [pltpu.VMEM((_BPW, D), jnp.float32) for _ in range(4)]
        + [pltpu.SemaphoreType.DMA for _ in range(4)]
    ),
)
def _gather_b(i_c, i_cs, i_dp, i_bt, t_c, t_cs, t_dp, t_bt, order_token,
              out_hbm, x0, x1, x2, x3, r0, r1, r2, r3, s0, s1, s2, s3):
    del order_token
    wid = lax.axis_index("s") * _NC + lax.axis_index("c")
    base = wid * _BPW
    idxs = (i_c, i_cs, i_dp, i_bt)
    tabs = (t_c, t_cs, t_dp, t_bt)
    ivs = (x0, x1, x2, x3)
    rvs = (r0, r1, r2, r3)
    sems = (s0, s1, s2, s3)
    for f in range(4):
        _stage_idx(idxs[f], ivs[f], base)
    for f in range(4):
        _fire_rows(tabs[f], ivs[f], rvs[f], sems[f])
    for f in range(4):
        _drain_rows(out_hbm.at[f, pl.ds(base, _BPW)], rvs[f], sems[f])


_BLK = 1024


def _proj_body(emb_o_ref, emb_b_ref, w_ref, b_ref, out_ref):
    x_o = emb_o_ref[...]
    eb = emb_b_ref[...]
    x_c, x_cs, x_dp, x_bt = eb[0], eb[1], eb[2], eb[3]
    w = w_ref[...]
    bias = b_ref[...]
    x_cs_t = jnp.swapaxes(x_cs, 0, 1)
    x_dp_t = jnp.swapaxes(x_dp, 0, 1)
    x_bt_t = jnp.swapaxes(x_bt, 0, 1)
    cdims = (((1,), (1,)), ((), ()))
    y_c_t = (lax.dot_general(w[0], x_c, cdims,
                             preferred_element_type=jnp.float32)
             + bias[0][:, None] + x_cs_t)
    y_o_t = (lax.dot_general(w[1], x_o, cdims,
                             preferred_element_type=jnp.float32)
             + bias[1][:, None] + x_dp_t)
    y_dp_t = (lax.dot_general(w[2], x_dp, cdims,
                              preferred_element_type=jnp.float32)
              + bias[2][:, None] + x_bt_t)
    out_ref[...] = jnp.concatenate([y_c_t, x_cs_t, y_o_t, y_dp_t, x_bt_t],
                                   axis=0)


_proj = pl.pallas_call(
    _proj_body,
    grid=(B // _BLK,),
    in_specs=[
        pl.BlockSpec((_BLK, D), lambda i: (i, 0)),
        pl.BlockSpec((4, _BLK, D), lambda i: (0, i, 0)),
        pl.BlockSpec((3, D, D), lambda i: (0, 0, 0)),
        pl.BlockSpec((3, D), lambda i: (0, 0)),
    ],
    out_specs=pl.BlockSpec((5 * D, _BLK), lambda i: (0, i)),
    out_shape=jax.ShapeDtypeStruct((5 * D, B), jnp.float32),
)


def kernel(campaignid, campaignsetid, offerid, demand_pkgname, business_type,
           table_campaignid, table_campaignsetid, table_offerid,
           table_demand_pkgname, table_business_type,
           W_campaignid, b_campaignid, W_offerid, b_offerid,
           W_demand_pkgname, b_demand_pkgname):
    i_c = campaignid.astype(jnp.int32)
    i_cs = campaignsetid.astype(jnp.int32)
    i_o = offerid.astype(jnp.int32)
    i_dp = demand_pkgname.astype(jnp.int32)
    i_bt = business_type.astype(jnp.int32)
    emb_o = _gather_a(i_o, table_offerid.T)
    emb_b = _gather_b(i_c, i_cs, i_dp, i_bt,
                      table_campaignid, table_campaignsetid,
                      table_demand_pkgname, table_business_type, emb_o)
    w = jnp.stack([W_campaignid, W_offerid, W_demand_pkgname])
    bias = jnp.stack([b_campaignid, b_offerid, b_demand_pkgname])
    return _proj(emb_o, emb_b, w, bias).T


# speedup vs baseline: 1.4679x; 1.0111x over previous
import functools

import jax
import jax.numpy as jnp
from jax import lax
from jax.experimental import pallas as pl
from jax.experimental.pallas import tpu as pltpu
from jax.experimental.pallas import tpu_sc as plsc

D = 64
B = 4096
_STRIPE = 128
_NSB = 8

_info = plsc.get_sparse_core_info()
_NC = _info.num_cores
_NS = _info.num_subcores
_NW = _NC * _NS
_BPW = B // _NW

_mesh = plsc.VectorSubcoreMesh(core_axis_name="c", subcore_axis_name="s")


def _stage_idx(idx_hbm, iv, base):
    pltpu.sync_copy(idx_hbm.at[pl.ds(base, _BPW)], iv)


def _fire_rows(tab, iv, rv, sem):
    def body(g, carry):
        v = iv[pl.ds(g * 16, 16)]
        for j in range(16):
            row = v[j]
            pltpu.async_copy(tab.at[row], rv.at[g * 16 + j], sem)
        return carry

    lax.fori_loop(0, _BPW // 16, body, 0)


def _drain_rows(out_slice, rv, sem):
    pltpu.make_async_copy(out_slice, rv, sem).wait()
    pltpu.sync_copy(rv, out_slice)


@functools.partial(
    pl.kernel,
    mesh=_mesh,
    compiler_params=pltpu.CompilerParams(needs_layout_passes=False),
    out_type=jax.ShapeDtypeStruct((B, D), jnp.float32),
    scratch_types=(
        [pltpu.VMEM((_BPW,), jnp.int32),
         pltpu.VMEM((_BPW, D), jnp.float32)]
        + [pltpu.VMEM((D, _STRIPE), jnp.float32) for _ in range(_NSB)]
        + [pltpu.SemaphoreType.DMA for _ in range(_NSB)]
    ),
)
def _gather_a(i_o, t_ot, out_hbm, iv, rv, sb0, sb1, sb2, sb3, sb4, sb5, sb6,
              sb7, q0, q1, q2, q3, q4, q5, q6, q7):
    wid = lax.axis_index("s") * _NC + lax.axis_index("c")
    base = wid * _BPW
    sbufs = (sb0, sb1, sb2, sb3, sb4, sb5, sb6, sb7)
    qsems = (q0, q1, q2, q3, q4, q5, q6, q7)
    _stage_idx(i_o, iv, base)

    jvecs = [lax.iota(jnp.int32, 16) + 16 * k for k in range(4)]

    def _extract(lane, buf, i):
        lvec = jnp.full((16,), lane, dtype=jnp.int32)
        for k in range(4):
            col = plsc.load_gather(buf, [jvecs[k], lvec])
            rv[i, pl.ds(k * 16, 16)] = col

    def _stripe_body(g, carry):
        v = iv[pl.ds(g * 16, 16)]
        pend = []
        for j in range(16):
            row = v[j]
            base_lane = pl.multiple_of((row // _STRIPE) * _STRIPE, _STRIPE)
            lane = row - base_lane
            nb = j % _NSB
            if j >= _NSB:
                plane, pcopy = pend[j - _NSB]
                pcopy.wait()
                _extract(plane, sbufs[nb], g * 16 + (j - _NSB))
            cp = pltpu.async_copy(
                t_ot.at[:, pl.ds(base_lane, _STRIPE)], sbufs[nb], qsems[nb])
            pend.append((lane, cp))
        for j in range(16 - _NSB, 16):
            plane, pcopy = pend[j]
            pcopy.wait()
            _extract(plane, sbufs[j % _NSB], g * 16 + j)
        return carry

    lax.fori_loop(0, _BPW // 16, _stripe_body, 0)
    pltpu.sync_copy(rv, out_hbm.at[pl.ds(base, _BPW)])


@functools.partial(
    pl.kernel,
    mesh=_mesh,
    compiler_params=pltpu.CompilerParams(needs_layout_passes=False),
    out_type=jax.ShapeDtypeStruct((4, B, D), jnp.float32),
    scratch_types=(
        [pltpu.VMEM((_BPW,), jnp.int32) for _ in range(4)]
        + [pltpu.VMEM((_BPW, D), jnp.float32) for _ in range(4)]
        + [pltpu.SemaphoreType.DMA for _ in range(4)]
    ),
)
def _gather_b(i_c, i_cs, i_dp, i_bt, t_c, t_cs, t_dp, t_bt, order_token,
              out_hbm, x0, x1, x2, x3, r0, r1, r2, r3, s0, s1, s2, s3):
    del order_token
    wid = lax.axis_index("s") * _NC + lax.axis_index("c")
    base = wid * _BPW
    idxs = (i_c, i_cs, i_dp, i_bt)
    tabs = (t_c, t_cs, t_dp, t_bt)
    ivs = (x0, x1, x2, x3)
    rvs = (r0, r1, r2, r3)
    sems = (s0, s1, s2, s3)
    for f in range(4):
        _stage_idx(idxs[f], ivs[f], base)
    for f in range(4):
        _fire_rows(tabs[f], ivs[f], rvs[f], sems[f])
    for f in range(4):
        _drain_rows(out_hbm.at[f, pl.ds(base, _BPW)], rvs[f], sems[f])


_BLK = 2048


def _proj_body(emb_o_ref, emb_b_ref, w_ref, b_ref, out_ref):
    x_o = emb_o_ref[...]
    eb = emb_b_ref[...]
    x_c, x_cs, x_dp, x_bt = eb[0], eb[1], eb[2], eb[3]
    w = w_ref[...]
    bias = b_ref[...]
    x_cs_t = jnp.swapaxes(x_cs, 0, 1)
    x_dp_t = jnp.swapaxes(x_dp, 0, 1)
    x_bt_t = jnp.swapaxes(x_bt, 0, 1)
    cdims = (((1,), (1,)), ((), ()))
    y_c_t = (lax.dot_general(w[0], x_c, cdims,
                             preferred_element_type=jnp.float32)
             + bias[0][:, None] + x_cs_t)
    y_o_t = (lax.dot_general(w[1], x_o, cdims,
                             preferred_element_type=jnp.float32)
             + bias[1][:, None] + x_dp_t)
    y_dp_t = (lax.dot_general(w[2], x_dp, cdims,
                              preferred_element_type=jnp.float32)
              + bias[2][:, None] + x_bt_t)
    out_ref[...] = jnp.concatenate([y_c_t, x_cs_t, y_o_t, y_dp_t, x_bt_t],
                                   axis=0)


_proj = pl.pallas_call(
    _proj_body,
    grid=(B // _BLK,),
    in_specs=[
        pl.BlockSpec((_BLK, D), lambda i: (i, 0)),
        pl.BlockSpec((4, _BLK, D), lambda i: (0, i, 0)),
        pl.BlockSpec((3, D, D), lambda i: (0, 0, 0)),
        pl.BlockSpec((3, D), lambda i: (0, 0)),
    ],
    out_specs=pl.BlockSpec((5 * D, _BLK), lambda i: (0, i)),
    out_shape=jax.ShapeDtypeStruct((5 * D, B), jnp.float32),
)


def kernel(campaignid, campaignsetid, offerid, demand_pkgname, business_type,
           table_campaignid, table_campaignsetid, table_offerid,
           table_demand_pkgname, table_business_type,
           W_campaignid, b_campaignid, W_offerid, b_offerid,
           W_demand_pkgname, b_demand_pkgname):
    i_c = campaignid.astype(jnp.int32)
    i_cs = campaignsetid.astype(jnp.int32)
    i_o = offerid.astype(jnp.int32)
    i_dp = demand_pkgname.astype(jnp.int32)
    i_bt = business_type.astype(jnp.int32)
    emb_o = _gather_a(i_o, table_offerid.T)
    emb_b = _gather_b(i_c, i_cs, i_dp, i_bt,
                      table_campaignid, table_campaignsetid,
                      table_demand_pkgname, table_business_type, emb_o)
    w = jnp.stack([W_campaignid, W_offerid, W_demand_pkgname])
    bias = jnp.stack([b_campaignid, b_offerid, b_demand_pkgname])
    return _proj(emb_o, emb_b, w, bias).T
